# Initial kernel scaffold; baseline (speedup 1.0000x reference)
import jax
import jax.numpy as jnp
from jax.experimental import pallas as pl


def kernel(x, edge_index, W1, b1, W2, b2, W3, b3):
    raise NotImplementedError("write your pallas kernel here")


# speedup vs baseline: 14.4835x; 14.4835x over previous
import functools

import jax
import jax.numpy as jnp
from jax import lax
from jax.experimental import pallas as pl
from jax.experimental.pallas import tpu as pltpu
from jax.experimental.pallas import tpu_sc as plsc

NC = 2
NS = 16
NW = NC * NS
CH = 128
DEGW = 128


def _sc_mesh():
    return plsc.VectorSubcoreMesh(
        core_axis_name="c", subcore_axis_name="s", num_cores=NC, num_subcores=NS
    )


def _part_copy(sid, n, copy_slice):
    rpt = (n // NS) // 8 * 8
    rem = n - NS * rpt
    copy_slice(sid * rpt, rpt)
    if rem:
        @pl.when(sid == 0)
        def _():
            copy_slice(NS * rpt, rem)


def _deg_body(dst_hbm, ones_hbm, zeros_hbm, out_hbm, ones_v, idx_v, deg_sh):
    cid = lax.axis_index("c")
    sid = lax.axis_index("s")
    wid = cid * NS + sid
    n = deg_sh.shape[0]
    _part_copy(sid, n, lambda o, sz: pltpu.sync_copy(
        zeros_hbm.at[pl.ds(o, sz)], deg_sh.at[pl.ds(o, sz)]))
    pltpu.sync_copy(ones_hbm, ones_v)
    plsc.subcore_barrier()

    nchunks = dst_hbm.shape[0] // CH

    def body(i, carry):
        c = wid + i * NW

        @pl.when(c < nchunks)
        def _():
            pltpu.sync_copy(dst_hbm.at[pl.ds(c * CH, CH)], idx_v)
            pltpu.sync_copy(ones_v, deg_sh.at[idx_v], add=True)

        return carry

    lax.fori_loop(0, (nchunks + NW - 1) // NW, body, 0)
    plsc.subcore_barrier()
    _part_copy(sid, n, lambda o, sz: pltpu.sync_copy(
        deg_sh.at[pl.ds(o, sz)], out_hbm.at[cid, pl.ds(o, sz)]))


def _agg_body(src_hbm, dst_hbm, g_hbm, zeros_hbm, out_hbm,
              src_v, dst_v, rows_v, sem, acc_sh):
    cid = lax.axis_index("c")
    sid = lax.axis_index("s")
    wid = cid * NS + sid
    n = acc_sh.shape[0]
    _part_copy(sid, n, lambda o, sz: pltpu.sync_copy(
        zeros_hbm.at[pl.ds(o, sz)], acc_sh.at[pl.ds(o, sz)]))
    plsc.subcore_barrier()

    nchunks = src_hbm.shape[0] // CH

    def body(i, carry):
        c = wid + i * NW

        @pl.when(c < nchunks)
        def _():
            pltpu.sync_copy(src_hbm.at[pl.ds(c * CH, CH)], src_v)
            pltpu.sync_copy(dst_hbm.at[pl.ds(c * CH, CH)], dst_v)
            pltpu.async_copy(g_hbm.at[src_v], rows_v, sem).wait()
            pltpu.sync_copy(rows_v, acc_sh.at[dst_v], add=True)

        return carry

    lax.fori_loop(0, (nchunks + NW - 1) // NW, body, 0)
    plsc.subcore_barrier()
    _part_copy(sid, n, lambda o, sz: pltpu.sync_copy(
        acc_sh.at[pl.ds(o, sz)], out_hbm.at[cid, pl.ds(o, sz)]))


def _sc_degree(dst, ones, zeros, n):
    call = pl.kernel(
        _deg_body,
        out_type=jax.ShapeDtypeStruct((NC, n, DEGW), jnp.float32),
        mesh=_sc_mesh(),
        scratch_types=[
            pltpu.VMEM((CH, DEGW), jnp.float32),
            pltpu.VMEM((CH,), jnp.int32),
            pltpu.VMEM_SHARED((n, DEGW), jnp.float32),
        ],
    )
    return call(dst, ones, zeros)


def _sc_aggregate(src, dst, g, zeros, n, d):
    call = pl.kernel(
        _agg_body,
        out_type=jax.ShapeDtypeStruct((NC, n, d), jnp.float32),
        mesh=_sc_mesh(),
        scratch_types=[
            pltpu.VMEM((CH,), jnp.int32),
            pltpu.VMEM((CH,), jnp.int32),
            pltpu.VMEM((CH, d), jnp.float32),
            pltpu.SemaphoreType.DMA,
            pltpu.VMEM_SHARED((n, d), jnp.float32),
        ],
    )
    return call(src, dst, g, zeros)


def _dinv(d0_ref, d1_ref):
    deg = d0_ref[:, 0:1] + d1_ref[:, 0:1] + 1.0
    return lax.rsqrt(deg)


def _lin1_body(x_ref, w_ref, d0_ref, d1_ref, g_ref):
    h = jnp.dot(x_ref[...], w_ref[...], preferred_element_type=jnp.float32)
    g_ref[...] = _dinv(d0_ref, d1_ref) * h


def _mid_body(a0_ref, a1_ref, g_ref, d0_ref, d1_ref, b_ref, w_ref, o_ref):
    dinv = _dinv(d0_ref, d1_ref)
    h = dinv * (a0_ref[...] + a1_ref[...] + g_ref[...]) + b_ref[...]
    h = jnp.maximum(h, 0.0)
    o_ref[...] = dinv * jnp.dot(h, w_ref[...], preferred_element_type=jnp.float32)


def _out_body(a0_ref, a1_ref, g_ref, d0_ref, d1_ref, b_ref, w_ref, b3_ref, o_ref):
    dinv = _dinv(d0_ref, d1_ref)
    h = dinv * (a0_ref[...] + a1_ref[...] + g_ref[...]) + b_ref[...]
    h = jnp.maximum(h, 0.0)
    o_ref[...] = (
        jnp.dot(h, w_ref[...], preferred_element_type=jnp.float32) + b3_ref[...]
    )


def _row_spec(br, d):
    return pl.BlockSpec((br, d), lambda i: (i, 0))


def _rep_spec(shape):
    return pl.BlockSpec(shape, lambda i: (0,) * len(shape))


def _tc_lin1(x, w, d0, d1, br):
    n, d = x.shape
    h = w.shape[1]
    return pl.pallas_call(
        _lin1_body,
        grid=(n // br,),
        in_specs=[
            _row_spec(br, d),
            _rep_spec(w.shape),
            _row_spec(br, DEGW),
            _row_spec(br, DEGW),
        ],
        out_specs=_row_spec(br, h),
        out_shape=jax.ShapeDtypeStruct((n, h), jnp.float32),
    )(x, w, d0, d1)


def _tc_mid(a0, a1, g, d0, d1, b, w, br):
    n, h = g.shape
    h2 = w.shape[1]
    return pl.pallas_call(
        _mid_body,
        grid=(n // br,),
        in_specs=[
            _row_spec(br, h),
            _row_spec(br, h),
            _row_spec(br, h),
            _row_spec(br, DEGW),
            _row_spec(br, DEGW),
            _rep_spec(b.shape),
            _rep_spec(w.shape),
        ],
        out_specs=_row_spec(br, h2),
        out_shape=jax.ShapeDtypeStruct((n, h2), jnp.float32),
    )(a0, a1, g, d0, d1, b, w)


def _tc_out(a0, a1, g, d0, d1, b, w, b3, br):
    n, h = g.shape
    c = w.shape[1]
    return pl.pallas_call(
        _out_body,
        grid=(n // br,),
        in_specs=[
            _row_spec(br, h),
            _row_spec(br, h),
            _row_spec(br, h),
            _row_spec(br, DEGW),
            _row_spec(br, DEGW),
            _rep_spec(b.shape),
            _rep_spec(w.shape),
            _rep_spec(b3.shape),
        ],
        out_specs=_row_spec(br, c),
        out_shape=jax.ShapeDtypeStruct((n, c), jnp.float32),
    )(a0, a1, g, d0, d1, b, w, b3)


@jax.jit
def kernel(x, edge_index, W1, b1, W2, b2, W3, b3):
    n, d = x.shape
    h = W1.shape[1]
    c = W3.shape[1]
    e = edge_index.shape[1]
    assert n % NS == 0 and e % CH == 0 and d % 128 == 0 and h % 128 == 0

    br = 2000 if n % 2000 == 0 else max(
        bb for bb in (8, 16, 40, 200, 400, 1000) if n % bb == 0
    )

    src = edge_index[0].astype(jnp.int32)
    dst = edge_index[1].astype(jnp.int32)

    ones = jnp.ones((CH, DEGW), jnp.float32)
    zeros_acc = jnp.zeros((n, h), jnp.float32)

    degw = _sc_degree(dst, ones, zeros_acc, n)
    d0, d1 = degw[0], degw[1]

    b1r = b1.reshape(1, h)
    b2r = b2.reshape(1, h)

    wpad = 128
    W3p = jnp.zeros((h, wpad), jnp.float32).at[:, :c].set(W3)
    b3p = jnp.zeros((1, wpad), jnp.float32).at[:, :c].set(b3)

    g1 = _tc_lin1(x, W1, d0, d1, br)
    acc1 = _sc_aggregate(src, dst, g1, zeros_acc, n, h)
    g2 = _tc_mid(acc1[0], acc1[1], g1, d0, d1, b1r, W2, br)
    acc2 = _sc_aggregate(src, dst, g2, zeros_acc, n, h)
    outp = _tc_out(acc2[0], acc2[1], g2, d0, d1, b2r, W3p, b3p, br)
    return outp[:, :c]


# speedup vs baseline: 21.6107x; 1.4921x over previous
import functools

import jax
import jax.numpy as jnp
from jax import lax
from jax.experimental import pallas as pl
from jax.experimental.pallas import tpu as pltpu
from jax.experimental.pallas import tpu_sc as plsc

NC = 2
NS = 16
NW = NC * NS
CH = 128
DEGW = 128


def _sc_mesh():
    return plsc.VectorSubcoreMesh(
        core_axis_name="c", subcore_axis_name="s", num_cores=NC, num_subcores=NS
    )


def _part_copy(sid, n, copy_slice):
    rpt = (n // NS) // 8 * 8
    rem = n - NS * rpt
    copy_slice(sid * rpt, rpt)
    if rem:
        @pl.when(sid == 0)
        def _():
            copy_slice(NS * rpt, rem)


def _deg_body(dst_hbm, ones_hbm, zeros_hbm, out_hbm,
              ones_v, idx0_v, idx1_v, sem, deg_sh):
    cid = lax.axis_index("c")
    sid = lax.axis_index("s")
    wid = cid * NS + sid
    n = deg_sh.shape[0]
    _part_copy(sid, n, lambda o, sz: pltpu.sync_copy(
        zeros_hbm.at[pl.ds(o, sz)], deg_sh.at[pl.ds(o, sz)]))
    pltpu.sync_copy(ones_hbm, ones_v)
    plsc.subcore_barrier()

    nchunks = dst_hbm.shape[0] // CH

    def load(c, idx_v):
        pltpu.sync_copy(dst_hbm.at[pl.ds(c * CH, CH)], idx_v)

    def sstart(idx_v):
        pltpu.async_copy(ones_v, deg_sh.at[idx_v], sem, add=True)

    def swait(idx_v):
        pltpu.make_async_copy(ones_v, deg_sh.at[idx_v], sem).wait()

    load(wid, idx0_v)
    sstart(idx0_v)

    def body(i, carry):
        c0 = wid + (2 * i) * NW
        c1 = c0 + NW
        c2 = c1 + NW

        @pl.when(c1 < nchunks)
        def _():
            load(c1, idx1_v)
            sstart(idx1_v)

        @pl.when(c0 < nchunks)
        def _():
            swait(idx0_v)

        @pl.when(c2 < nchunks)
        def _():
            load(c2, idx0_v)
            sstart(idx0_v)

        @pl.when(c1 < nchunks)
        def _():
            swait(idx1_v)

        return carry

    lax.fori_loop(0, (nchunks // NW + 2) // 2, body, 0)
    plsc.subcore_barrier()
    _part_copy(sid, n, lambda o, sz: pltpu.sync_copy(
        deg_sh.at[pl.ds(o, sz)], out_hbm.at[cid, pl.ds(o, sz)]))


def _agg_body(src_hbm, dst_hbm, g_hbm, zeros_hbm, out_hbm,
              src0_v, dst0_v, rows0_v, sem0,
              src1_v, dst1_v, rows1_v, sem1, acc_sh):
    cid = lax.axis_index("c")
    sid = lax.axis_index("s")
    wid = cid * NS + sid
    n = acc_sh.shape[0]
    _part_copy(sid, n, lambda o, sz: pltpu.sync_copy(
        zeros_hbm.at[pl.ds(o, sz)], acc_sh.at[pl.ds(o, sz)]))
    plsc.subcore_barrier()

    nchunks = src_hbm.shape[0] // CH
    bufs = ((src0_v, dst0_v, rows0_v, sem0), (src1_v, dst1_v, rows1_v, sem1))

    def gstart(c, b):
        src_v, dst_v, rows_v, sem = bufs[b]
        pltpu.sync_copy(src_hbm.at[pl.ds(c * CH, CH)], src_v)
        pltpu.sync_copy(dst_hbm.at[pl.ds(c * CH, CH)], dst_v)
        pltpu.async_copy(g_hbm.at[src_v], rows_v, sem)

    def finish(b):
        src_v, dst_v, rows_v, sem = bufs[b]
        pltpu.make_async_copy(g_hbm.at[src_v], rows_v, sem).wait()
        pltpu.sync_copy(rows_v, acc_sh.at[dst_v], add=True)

    gstart(wid, 0)

    def body(i, carry):
        c0 = wid + (2 * i) * NW
        c1 = c0 + NW
        c2 = c1 + NW

        @pl.when(c1 < nchunks)
        def _():
            gstart(c1, 1)

        @pl.when(c0 < nchunks)
        def _():
            finish(0)

        @pl.when(c2 < nchunks)
        def _():
            gstart(c2, 0)

        @pl.when(c1 < nchunks)
        def _():
            finish(1)

        return carry

    lax.fori_loop(0, (nchunks // NW + 2) // 2, body, 0)
    plsc.subcore_barrier()
    _part_copy(sid, n, lambda o, sz: pltpu.sync_copy(
        acc_sh.at[pl.ds(o, sz)], out_hbm.at[cid, pl.ds(o, sz)]))


def _sc_degree(dst, ones, zeros, n):
    call = pl.kernel(
        _deg_body,
        out_type=jax.ShapeDtypeStruct((NC, n, DEGW), jnp.float32),
        mesh=_sc_mesh(),
        scratch_types=[
            pltpu.VMEM((CH, DEGW), jnp.float32),
            pltpu.VMEM((CH,), jnp.int32),
            pltpu.VMEM((CH,), jnp.int32),
            pltpu.SemaphoreType.DMA,
            pltpu.VMEM_SHARED((n, DEGW), jnp.float32),
        ],
    )
    return call(dst, ones, zeros)


def _sc_aggregate(src, dst, g, zeros, n, d):
    call = pl.kernel(
        _agg_body,
        out_type=jax.ShapeDtypeStruct((NC, n, d), jnp.float32),
        mesh=_sc_mesh(),
        scratch_types=[
            pltpu.VMEM((CH,), jnp.int32),
            pltpu.VMEM((CH,), jnp.int32),
            pltpu.VMEM((CH, d), jnp.float32),
            pltpu.SemaphoreType.DMA,
            pltpu.VMEM((CH,), jnp.int32),
            pltpu.VMEM((CH,), jnp.int32),
            pltpu.VMEM((CH, d), jnp.float32),
            pltpu.SemaphoreType.DMA,
            pltpu.VMEM_SHARED((n, d), jnp.float32),
        ],
    )
    return call(src, dst, g, zeros)


def _dinv(d0_ref, d1_ref):
    deg = d0_ref[:, 0:1] + d1_ref[:, 0:1] + 1.0
    return lax.rsqrt(deg)


def _lin1_body(x_ref, w_ref, d0_ref, d1_ref, g_ref):
    h = jnp.dot(x_ref[...], w_ref[...], preferred_element_type=jnp.float32)
    g_ref[...] = _dinv(d0_ref, d1_ref) * h


def _mid_body(a0_ref, a1_ref, g_ref, d0_ref, d1_ref, b_ref, w_ref, o_ref):
    dinv = _dinv(d0_ref, d1_ref)
    h = dinv * (a0_ref[...] + a1_ref[...] + g_ref[...]) + b_ref[...]
    h = jnp.maximum(h, 0.0)
    o_ref[...] = dinv * jnp.dot(h, w_ref[...], preferred_element_type=jnp.float32)


def _out_body(a0_ref, a1_ref, g_ref, d0_ref, d1_ref, b_ref, w_ref, b3_ref, o_ref):
    dinv = _dinv(d0_ref, d1_ref)
    h = dinv * (a0_ref[...] + a1_ref[...] + g_ref[...]) + b_ref[...]
    h = jnp.maximum(h, 0.0)
    o_ref[...] = (
        jnp.dot(h, w_ref[...], preferred_element_type=jnp.float32) + b3_ref[...]
    )


def _row_spec(br, d):
    return pl.BlockSpec((br, d), lambda i: (i, 0))


def _rep_spec(shape):
    return pl.BlockSpec(shape, lambda i: (0,) * len(shape))


def _tc_lin1(x, w, d0, d1, br):
    n, d = x.shape
    h = w.shape[1]
    return pl.pallas_call(
        _lin1_body,
        grid=(n // br,),
        in_specs=[
            _row_spec(br, d),
            _rep_spec(w.shape),
            _row_spec(br, DEGW),
            _row_spec(br, DEGW),
        ],
        out_specs=_row_spec(br, h),
        out_shape=jax.ShapeDtypeStruct((n, h), jnp.float32),
    )(x, w, d0, d1)


def _tc_mid(a0, a1, g, d0, d1, b, w, br):
    n, h = g.shape
    h2 = w.shape[1]
    return pl.pallas_call(
        _mid_body,
        grid=(n // br,),
        in_specs=[
            _row_spec(br, h),
            _row_spec(br, h),
            _row_spec(br, h),
            _row_spec(br, DEGW),
            _row_spec(br, DEGW),
            _rep_spec(b.shape),
            _rep_spec(w.shape),
        ],
        out_specs=_row_spec(br, h2),
        out_shape=jax.ShapeDtypeStruct((n, h2), jnp.float32),
    )(a0, a1, g, d0, d1, b, w)


def _tc_out(a0, a1, g, d0, d1, b, w, b3, br):
    n, h = g.shape
    c = w.shape[1]
    return pl.pallas_call(
        _out_body,
        grid=(n // br,),
        in_specs=[
            _row_spec(br, h),
            _row_spec(br, h),
            _row_spec(br, h),
            _row_spec(br, DEGW),
            _row_spec(br, DEGW),
            _rep_spec(b.shape),
            _rep_spec(w.shape),
            _rep_spec(b3.shape),
        ],
        out_specs=_row_spec(br, c),
        out_shape=jax.ShapeDtypeStruct((n, c), jnp.float32),
    )(a0, a1, g, d0, d1, b, w, b3)


@jax.jit
def kernel(x, edge_index, W1, b1, W2, b2, W3, b3):
    n, d = x.shape
    h = W1.shape[1]
    c = W3.shape[1]
    e = edge_index.shape[1]
    assert n % NS == 0 and e % CH == 0 and d % 128 == 0 and h % 128 == 0

    br = 2000 if n % 2000 == 0 else max(
        bb for bb in (8, 16, 40, 200, 400, 1000) if n % bb == 0
    )

    src = edge_index[0].astype(jnp.int32)
    dst = edge_index[1].astype(jnp.int32)

    ones = jnp.ones((CH, DEGW), jnp.float32)
    zeros_acc = jnp.zeros((n, h), jnp.float32)

    degw = _sc_degree(dst, ones, zeros_acc, n)
    d0, d1 = degw[0], degw[1]

    b1r = b1.reshape(1, h)
    b2r = b2.reshape(1, h)

    wpad = 128
    W3p = jnp.zeros((h, wpad), jnp.float32).at[:, :c].set(W3)
    b3p = jnp.zeros((1, wpad), jnp.float32).at[:, :c].set(b3)

    g1 = _tc_lin1(x, W1, d0, d1, br)
    acc1 = _sc_aggregate(src, dst, g1, zeros_acc, n, h)
    g2 = _tc_mid(acc1[0], acc1[1], g1, d0, d1, b1r, W2, br)
    acc2 = _sc_aggregate(src, dst, g2, zeros_acc, n, h)
    outp = _tc_out(acc2[0], acc2[1], g2, d0, d1, b2r, W3p, b3p, br)
    return outp[:, :c]


# speedup vs baseline: 21.6999x; 1.0041x over previous
import functools

import jax
import jax.numpy as jnp
from jax import lax
from jax.experimental import pallas as pl
from jax.experimental.pallas import tpu as pltpu
from jax.experimental.pallas import tpu_sc as plsc

NC = 2
NS = 16
NW = NC * NS
CH = 128
DEGW = 128


def _sc_mesh():
    return plsc.VectorSubcoreMesh(
        core_axis_name="c", subcore_axis_name="s", num_cores=NC, num_subcores=NS
    )


def _part_copy(sid, n, copy_slice):
    rpt = (n // NS) // 8 * 8
    rem = n - NS * rpt
    copy_slice(sid * rpt, rpt)
    if rem:
        @pl.when(sid == 0)
        def _():
            copy_slice(NS * rpt, rem)


def _deg_body(dst_hbm, ones_hbm, zeros_hbm, out_hbm,
              ones_v, idx0_v, idx1_v, sem, deg_sh):
    cid = lax.axis_index("c")
    sid = lax.axis_index("s")
    wid = cid * NS + sid
    n = deg_sh.shape[0]
    _part_copy(sid, n, lambda o, sz: pltpu.sync_copy(
        zeros_hbm.at[pl.ds(o, sz)], deg_sh.at[pl.ds(o, sz)]))
    pltpu.sync_copy(ones_hbm, ones_v)
    plsc.subcore_barrier()

    nchunks = dst_hbm.shape[0] // CH

    def load(c, idx_v):
        pltpu.sync_copy(dst_hbm.at[pl.ds(c * CH, CH)], idx_v)

    def sstart(idx_v):
        pltpu.async_copy(ones_v, deg_sh.at[idx_v], sem, add=True)

    def swait(idx_v):
        pltpu.make_async_copy(ones_v, deg_sh.at[idx_v], sem).wait()

    load(wid, idx0_v)
    sstart(idx0_v)

    def body(i, carry):
        c0 = wid + (2 * i) * NW
        c1 = c0 + NW
        c2 = c1 + NW

        @pl.when(c1 < nchunks)
        def _():
            load(c1, idx1_v)
            sstart(idx1_v)

        @pl.when(c0 < nchunks)
        def _():
            swait(idx0_v)

        @pl.when(c2 < nchunks)
        def _():
            load(c2, idx0_v)
            sstart(idx0_v)

        @pl.when(c1 < nchunks)
        def _():
            swait(idx1_v)

        return carry

    lax.fori_loop(0, (nchunks // NW + 2) // 2, body, 0)
    plsc.subcore_barrier()
    _part_copy(sid, n, lambda o, sz: pltpu.sync_copy(
        deg_sh.at[pl.ds(o, sz)], out_hbm.at[cid, pl.ds(o, sz)]))


def _agg_body(src_hbm, dst_hbm, g_hbm, zeros_hbm, out_hbm,
              src0_v, dst0_v, rows0_v, sem0,
              src1_v, dst1_v, rows1_v, sem1, acc_sh):
    cid = lax.axis_index("c")
    sid = lax.axis_index("s")
    wid = cid * NS + sid
    n = acc_sh.shape[0]
    _part_copy(sid, n, lambda o, sz: pltpu.sync_copy(
        zeros_hbm.at[pl.ds(o, sz)], acc_sh.at[pl.ds(o, sz)]))
    plsc.subcore_barrier()

    nchunks = src_hbm.shape[0] // CH
    bufs = ((src0_v, dst0_v, rows0_v, sem0), (src1_v, dst1_v, rows1_v, sem1))

    def gstart(c, b):
        src_v, dst_v, rows_v, sem = bufs[b]
        pltpu.sync_copy(src_hbm.at[pl.ds(c * CH, CH)], src_v)
        pltpu.sync_copy(dst_hbm.at[pl.ds(c * CH, CH)], dst_v)
        pltpu.async_copy(g_hbm.at[src_v], rows_v, sem)

    def finish(b):
        src_v, dst_v, rows_v, sem = bufs[b]
        pltpu.make_async_copy(g_hbm.at[src_v], rows_v, sem).wait()
        pltpu.sync_copy(rows_v, acc_sh.at[dst_v], add=True)

    gstart(wid, 0)

    def body(i, carry):
        c0 = wid + (2 * i) * NW
        c1 = c0 + NW
        c2 = c1 + NW

        @pl.when(c1 < nchunks)
        def _():
            gstart(c1, 1)

        @pl.when(c0 < nchunks)
        def _():
            finish(0)

        @pl.when(c2 < nchunks)
        def _():
            gstart(c2, 0)

        @pl.when(c1 < nchunks)
        def _():
            finish(1)

        return carry

    lax.fori_loop(0, (nchunks // NW + 2) // 2, body, 0)
    plsc.subcore_barrier()
    _part_copy(sid, n, lambda o, sz: pltpu.sync_copy(
        acc_sh.at[pl.ds(o, sz)], out_hbm.at[cid, pl.ds(o, sz)]))


def _sc_degree(dst, ones, zeros, n):
    call = pl.kernel(
        _deg_body,
        out_type=jax.ShapeDtypeStruct((NC, n, DEGW), jnp.float32),
        mesh=_sc_mesh(),
        scratch_types=[
            pltpu.VMEM((CH, DEGW), jnp.float32),
            pltpu.VMEM((CH,), jnp.int32),
            pltpu.VMEM((CH,), jnp.int32),
            pltpu.SemaphoreType.DMA,
            pltpu.VMEM_SHARED((n, DEGW), jnp.float32),
        ],
    )
    return call(dst, ones, zeros)


def _sc_aggregate(src, dst, g, zeros, n, d):
    call = pl.kernel(
        _agg_body,
        out_type=jax.ShapeDtypeStruct((NC, n, d), jnp.float32),
        mesh=_sc_mesh(),
        scratch_types=[
            pltpu.VMEM((CH,), jnp.int32),
            pltpu.VMEM((CH,), jnp.int32),
            pltpu.VMEM((CH, d), jnp.float32),
            pltpu.SemaphoreType.DMA,
            pltpu.VMEM((CH,), jnp.int32),
            pltpu.VMEM((CH,), jnp.int32),
            pltpu.VMEM((CH, d), jnp.float32),
            pltpu.SemaphoreType.DMA,
            pltpu.VMEM_SHARED((n, d), jnp.float32),
        ],
    )
    return call(src, dst, g, zeros)


def _mm_body(x_ref, w_ref, u_ref):
    u_ref[...] = jnp.dot(x_ref[...], w_ref[...], preferred_element_type=jnp.float32)


def _scale_body(u_ref, d0_ref, d1_ref, g_ref, dv_ref):
    deg = d0_ref[:, 0:1] + d1_ref[:, 0:1] + 1.0
    dinv = lax.rsqrt(deg)
    dv_ref[...] = jnp.broadcast_to(dinv, dv_ref.shape)
    g_ref[...] = dinv * u_ref[...]


def _mid_body(a0_ref, a1_ref, g_ref, dv_ref, b_ref, w_ref, o_ref):
    dinv = dv_ref[:, 0:1]
    h = dinv * (a0_ref[...] + a1_ref[...] + g_ref[...]) + b_ref[...]
    h = jnp.maximum(h, 0.0)
    o_ref[...] = dinv * jnp.dot(h, w_ref[...], preferred_element_type=jnp.float32)


def _out_body(a0_ref, a1_ref, g_ref, dv_ref, b_ref, w_ref, b3_ref, o_ref):
    dinv = dv_ref[:, 0:1]
    h = dinv * (a0_ref[...] + a1_ref[...] + g_ref[...]) + b_ref[...]
    h = jnp.maximum(h, 0.0)
    o_ref[...] = (
        jnp.dot(h, w_ref[...], preferred_element_type=jnp.float32) + b3_ref[...]
    )


def _row_spec(br, d):
    return pl.BlockSpec((br, d), lambda i: (i, 0))


def _rep_spec(shape):
    return pl.BlockSpec(shape, lambda i: (0,) * len(shape))


DVW = 8


def _tc_mm(x, w, br):
    n, d = x.shape
    h = w.shape[1]
    return pl.pallas_call(
        _mm_body,
        grid=(n // br,),
        in_specs=[_row_spec(br, d), _rep_spec(w.shape)],
        out_specs=_row_spec(br, h),
        out_shape=jax.ShapeDtypeStruct((n, h), jnp.float32),
    )(x, w)


def _tc_scale(u, d0, d1, br):
    n, h = u.shape
    return pl.pallas_call(
        _scale_body,
        grid=(n // br,),
        in_specs=[
            _row_spec(br, h),
            _row_spec(br, DEGW),
            _row_spec(br, DEGW),
        ],
        out_specs=[_row_spec(br, h), _row_spec(br, DVW)],
        out_shape=[
            jax.ShapeDtypeStruct((n, h), jnp.float32),
            jax.ShapeDtypeStruct((n, DVW), jnp.float32),
        ],
    )(u, d0, d1)


def _tc_mid(a0, a1, g, dv, b, w, br):
    n, h = g.shape
    h2 = w.shape[1]
    return pl.pallas_call(
        _mid_body,
        grid=(n // br,),
        in_specs=[
            _row_spec(br, h),
            _row_spec(br, h),
            _row_spec(br, h),
            _row_spec(br, DVW),
            _rep_spec(b.shape),
            _rep_spec(w.shape),
        ],
        out_specs=_row_spec(br, h2),
        out_shape=jax.ShapeDtypeStruct((n, h2), jnp.float32),
    )(a0, a1, g, dv, b, w)


def _tc_out(a0, a1, g, dv, b, w, b3, br):
    n, h = g.shape
    c = w.shape[1]
    return pl.pallas_call(
        _out_body,
        grid=(n // br,),
        in_specs=[
            _row_spec(br, h),
            _row_spec(br, h),
            _row_spec(br, h),
            _row_spec(br, DVW),
            _rep_spec(b.shape),
            _rep_spec(w.shape),
            _rep_spec(b3.shape),
        ],
        out_specs=_row_spec(br, c),
        out_shape=jax.ShapeDtypeStruct((n, c), jnp.float32),
    )(a0, a1, g, dv, b, w, b3)


@jax.jit
def kernel(x, edge_index, W1, b1, W2, b2, W3, b3):
    n, d = x.shape
    h = W1.shape[1]
    c = W3.shape[1]
    e = edge_index.shape[1]
    assert n % NS == 0 and e % CH == 0 and d % 128 == 0 and h % 128 == 0

    br = 2000 if n % 2000 == 0 else max(
        bb for bb in (8, 16, 40, 200, 400, 1000) if n % bb == 0
    )

    src = edge_index[0].astype(jnp.int32)
    dst = edge_index[1].astype(jnp.int32)

    ones = jnp.ones((CH, DEGW), jnp.float32)
    zeros_acc = jnp.zeros((n, h), jnp.float32)

    degw = _sc_degree(dst, ones, zeros_acc, n)
    u1 = _tc_mm(x, W1, br)

    b1r = b1.reshape(1, h)
    b2r = b2.reshape(1, h)

    wpad = 128
    W3p = jnp.zeros((h, wpad), jnp.float32).at[:, :c].set(W3)
    b3p = jnp.zeros((1, wpad), jnp.float32).at[:, :c].set(b3)

    g1, dv = _tc_scale(u1, degw[0], degw[1], br)
    acc1 = _sc_aggregate(src, dst, g1, zeros_acc, n, h)
    g2 = _tc_mid(acc1[0], acc1[1], g1, dv, b1r, W2, br)
    acc2 = _sc_aggregate(src, dst, g2, zeros_acc, n, h)
    outp = _tc_out(acc2[0], acc2[1], g2, dv, b2r, W3p, b3p, br)
    return outp[:, :c]


# speedup vs baseline: 23.2304x; 1.0705x over previous
import functools

import jax
import jax.numpy as jnp
from jax import lax
from jax.experimental import pallas as pl
from jax.experimental.pallas import tpu as pltpu
from jax.experimental.pallas import tpu_sc as plsc

NC = 2
NS = 16
NW = NC * NS
CH = 128
DEGW = 128


def _sc_mesh():
    return plsc.VectorSubcoreMesh(
        core_axis_name="c", subcore_axis_name="s", num_cores=NC, num_subcores=NS
    )


def _part_copy(sid, n, copy_slice):
    rpt = (n // NS) // 8 * 8
    rem = n - NS * rpt
    copy_slice(sid * rpt, rpt)
    if rem:
        @pl.when(sid == 0)
        def _():
            copy_slice(NS * rpt, rem)


def _deg_body(dst_hbm, ones_hbm, out_hbm,
              ones_v, idx0_v, idx1_v, slab_v, sem, deg_sh):
    cid = lax.axis_index("c")
    sid = lax.axis_index("s")
    wid = cid * NS + sid
    dpad = deg_sh.shape[0]
    spt = dpad // NS
    for i in range(spt // 16):
        slab_v[pl.ds(i * 16, 16)] = jnp.zeros((16,), jnp.float32)
    pltpu.sync_copy(slab_v, deg_sh.at[pl.ds(sid * spt, spt)])
    pltpu.sync_copy(ones_hbm, ones_v)
    plsc.subcore_barrier()

    nchunks = dst_hbm.shape[0] // CH

    def load(c, idx_v):
        pltpu.sync_copy(dst_hbm.at[pl.ds(c * CH, CH)], idx_v)

    def sstart(idx_v):
        pltpu.async_copy(ones_v, deg_sh.at[idx_v], sem, add=True)

    def swait(idx_v):
        pltpu.make_async_copy(ones_v, deg_sh.at[idx_v], sem).wait()

    load(wid, idx0_v)
    sstart(idx0_v)

    def body(i, carry):
        c0 = wid + (2 * i) * NW
        c1 = c0 + NW
        c2 = c1 + NW

        @pl.when(c1 < nchunks)
        def _():
            load(c1, idx1_v)
            sstart(idx1_v)

        @pl.when(c0 < nchunks)
        def _():
            swait(idx0_v)

        @pl.when(c2 < nchunks)
        def _():
            load(c2, idx0_v)
            sstart(idx0_v)

        @pl.when(c1 < nchunks)
        def _():
            swait(idx1_v)

        return carry

    lax.fori_loop(0, (nchunks // NW + 2) // 2, body, 0)
    plsc.subcore_barrier()
    pltpu.sync_copy(deg_sh.at[pl.ds(sid * spt, spt)], slab_v)
    pltpu.sync_copy(slab_v, out_hbm.at[pl.ds(cid * dpad + sid * spt, spt)])


def _agg_body(src_hbm, dst_hbm, g_hbm, zeros_hbm, out_hbm,
              src0_v, dst0_v, rows0_v, sem0,
              src1_v, dst1_v, rows1_v, sem1, acc_sh):
    cid = lax.axis_index("c")
    sid = lax.axis_index("s")
    wid = cid * NS + sid
    n = acc_sh.shape[0]
    _part_copy(sid, n, lambda o, sz: pltpu.sync_copy(
        zeros_hbm.at[pl.ds(o, sz)], acc_sh.at[pl.ds(o, sz)]))
    plsc.subcore_barrier()

    nchunks = src_hbm.shape[0] // CH
    bufs = ((src0_v, dst0_v, rows0_v, sem0), (src1_v, dst1_v, rows1_v, sem1))

    def gstart(c, b):
        src_v, dst_v, rows_v, sem = bufs[b]
        pltpu.sync_copy(src_hbm.at[pl.ds(c * CH, CH)], src_v)
        pltpu.sync_copy(dst_hbm.at[pl.ds(c * CH, CH)], dst_v)
        pltpu.async_copy(g_hbm.at[src_v], rows_v, sem)

    def finish(b):
        src_v, dst_v, rows_v, sem = bufs[b]
        pltpu.make_async_copy(g_hbm.at[src_v], rows_v, sem).wait()
        pltpu.sync_copy(rows_v, acc_sh.at[dst_v], add=True)

    gstart(wid, 0)

    def body(i, carry):
        c0 = wid + (2 * i) * NW
        c1 = c0 + NW
        c2 = c1 + NW

        @pl.when(c1 < nchunks)
        def _():
            gstart(c1, 1)

        @pl.when(c0 < nchunks)
        def _():
            finish(0)

        @pl.when(c2 < nchunks)
        def _():
            gstart(c2, 0)

        @pl.when(c1 < nchunks)
        def _():
            finish(1)

        return carry

    lax.fori_loop(0, (nchunks // NW + 2) // 2, body, 0)
    plsc.subcore_barrier()
    _part_copy(sid, n, lambda o, sz: pltpu.sync_copy(
        acc_sh.at[pl.ds(o, sz)], out_hbm.at[cid, pl.ds(o, sz)]))


def _sc_degree(dst, ones, dpad):
    call = pl.kernel(
        _deg_body,
        out_type=jax.ShapeDtypeStruct((NC * dpad,), jnp.float32),
        mesh=_sc_mesh(),
        scratch_types=[
            pltpu.VMEM((CH,), jnp.float32),
            pltpu.VMEM((CH,), jnp.int32),
            pltpu.VMEM((CH,), jnp.int32),
            pltpu.VMEM((dpad // NS,), jnp.float32),
            pltpu.SemaphoreType.DMA,
            pltpu.VMEM_SHARED((dpad,), jnp.float32),
        ],
    )
    return call(dst, ones)


def _sc_aggregate(src, dst, g, zeros, n, d):
    call = pl.kernel(
        _agg_body,
        out_type=jax.ShapeDtypeStruct((NC, n, d), jnp.float32),
        mesh=_sc_mesh(),
        scratch_types=[
            pltpu.VMEM((CH,), jnp.int32),
            pltpu.VMEM((CH,), jnp.int32),
            pltpu.VMEM((CH, d), jnp.float32),
            pltpu.SemaphoreType.DMA,
            pltpu.VMEM((CH,), jnp.int32),
            pltpu.VMEM((CH,), jnp.int32),
            pltpu.VMEM((CH, d), jnp.float32),
            pltpu.SemaphoreType.DMA,
            pltpu.VMEM_SHARED((n, d), jnp.float32),
        ],
    )
    return call(src, dst, g, zeros)


def _lin1_body(x_ref, w_ref, d0_ref, d1_ref, g_ref, dv_ref):
    deg = d0_ref[...] + d1_ref[...] + 1.0
    dinv = lax.rsqrt(deg)
    dv_ref[...] = jnp.broadcast_to(dinv, dv_ref.shape)
    u = jnp.dot(x_ref[...], w_ref[...], preferred_element_type=jnp.float32)
    g_ref[...] = dinv * u


def _mid_body(a0_ref, a1_ref, g_ref, dv_ref, b_ref, w_ref, o_ref):
    dinv = dv_ref[:, 0:1]
    h = dinv * (a0_ref[...] + a1_ref[...] + g_ref[...]) + b_ref[...]
    h = jnp.maximum(h, 0.0)
    o_ref[...] = dinv * jnp.dot(h, w_ref[...], preferred_element_type=jnp.float32)


def _out_body(a0_ref, a1_ref, g_ref, dv_ref, b_ref, w_ref, b3_ref, o_ref):
    dinv = dv_ref[:, 0:1]
    h = dinv * (a0_ref[...] + a1_ref[...] + g_ref[...]) + b_ref[...]
    h = jnp.maximum(h, 0.0)
    o_ref[...] = (
        jnp.dot(h, w_ref[...], preferred_element_type=jnp.float32) + b3_ref[...]
    )


def _row_spec(br, d):
    return pl.BlockSpec((br, d), lambda i: (i, 0))


def _rep_spec(shape):
    return pl.BlockSpec(shape, lambda i: (0,) * len(shape))


DVW = 8


def _tc_lin1(x, w, d0, d1, br):
    n, d = x.shape
    h = w.shape[1]
    return pl.pallas_call(
        _lin1_body,
        grid=(n // br,),
        in_specs=[
            _row_spec(br, d),
            _rep_spec(w.shape),
            _row_spec(br, 1),
            _row_spec(br, 1),
        ],
        out_specs=[_row_spec(br, h), _row_spec(br, DVW)],
        out_shape=[
            jax.ShapeDtypeStruct((n, h), jnp.float32),
            jax.ShapeDtypeStruct((n, DVW), jnp.float32),
        ],
    )(x, w, d0, d1)


def _tc_mid(a0, a1, g, dv, b, w, br):
    n, h = g.shape
    h2 = w.shape[1]
    return pl.pallas_call(
        _mid_body,
        grid=(n // br,),
        in_specs=[
            _row_spec(br, h),
            _row_spec(br, h),
            _row_spec(br, h),
            _row_spec(br, DVW),
            _rep_spec(b.shape),
            _rep_spec(w.shape),
        ],
        out_specs=_row_spec(br, h2),
        out_shape=jax.ShapeDtypeStruct((n, h2), jnp.float32),
    )(a0, a1, g, dv, b, w)


def _tc_out(a0, a1, g, dv, b, w, b3, br):
    n, h = g.shape
    c = w.shape[1]
    return pl.pallas_call(
        _out_body,
        grid=(n // br,),
        in_specs=[
            _row_spec(br, h),
            _row_spec(br, h),
            _row_spec(br, h),
            _row_spec(br, DVW),
            _rep_spec(b.shape),
            _rep_spec(w.shape),
            _rep_spec(b3.shape),
        ],
        out_specs=_row_spec(br, c),
        out_shape=jax.ShapeDtypeStruct((n, c), jnp.float32),
    )(a0, a1, g, dv, b, w, b3)


@jax.jit
def kernel(x, edge_index, W1, b1, W2, b2, W3, b3):
    n, d = x.shape
    h = W1.shape[1]
    c = W3.shape[1]
    e = edge_index.shape[1]
    assert n % NS == 0 and e % CH == 0 and d % 128 == 0 and h % 128 == 0

    br = 2000 if n % 2000 == 0 else max(
        bb for bb in (8, 16, 40, 200, 400, 1000) if n % bb == 0
    )

    src = edge_index[0].astype(jnp.int32)
    dst = edge_index[1].astype(jnp.int32)

    ones = jnp.ones((CH,), jnp.float32)
    zeros_acc = jnp.zeros((n, h), jnp.float32)

    spt = (-(-n // NS) + 15) // 16 * 16
    dpad = NS * spt

    degf = _sc_degree(dst, ones, dpad)
    d2 = degf.reshape(NC, dpad)
    d0 = d2[0, :n].reshape(n, 1)
    d1 = d2[1, :n].reshape(n, 1)

    b1r = b1.reshape(1, h)
    b2r = b2.reshape(1, h)

    wpad = 128
    W3p = jnp.zeros((h, wpad), jnp.float32).at[:, :c].set(W3)
    b3p = jnp.zeros((1, wpad), jnp.float32).at[:, :c].set(b3)

    g1, dv = _tc_lin1(x, W1, d0, d1, br)
    acc1 = _sc_aggregate(src, dst, g1, zeros_acc, n, h)
    g2 = _tc_mid(acc1[0], acc1[1], g1, dv, b1r, W2, br)
    acc2 = _sc_aggregate(src, dst, g2, zeros_acc, n, h)
    outp = _tc_out(acc2[0], acc2[1], g2, dv, b2r, W3p, b3p, br)
    return outp[:, :c]


# speedup vs baseline: 24.6552x; 1.0613x over previous
import functools

import jax
import jax.numpy as jnp
from jax import lax
from jax.experimental import pallas as pl
from jax.experimental.pallas import tpu as pltpu
from jax.experimental.pallas import tpu_sc as plsc

NC = 2
NS = 16
NW = NC * NS
CH = 128
DEGW = 128


def _sc_mesh():
    return plsc.VectorSubcoreMesh(
        core_axis_name="c", subcore_axis_name="s", num_cores=NC, num_subcores=NS
    )


def _part_copy(sid, n, copy_slice):
    rpt = (n // NS) // 8 * 8
    rem = n - NS * rpt
    copy_slice(sid * rpt, rpt)
    if rem:
        @pl.when(sid == 0)
        def _():
            copy_slice(NS * rpt, rem)


DG = 8


def _deg_body(dst_hbm, ones_hbm, out_hbm,
              ones_v, idxa_v, idxb_v, slab_v, sem, deg_sh):
    cid = lax.axis_index("c")
    sid = lax.axis_index("s")
    wid = cid * NS + sid
    dpad = deg_sh.shape[0]
    spt = dpad // NS
    for i in range(spt // 16):
        slab_v[pl.ds(i * 16, 16)] = jnp.zeros((16,), jnp.float32)
    pltpu.sync_copy(slab_v, deg_sh.at[pl.ds(sid * spt, spt)])
    pltpu.sync_copy(ones_hbm, ones_v)
    plsc.subcore_barrier()

    gpw = dst_hbm.shape[0] // (NW * DG)
    row0 = wid * gpw * DG

    def load(g, idx_v):
        pltpu.sync_copy(dst_hbm.at[pl.ds(row0 + g * DG, DG)], idx_v)

    def group(idx_v):
        for j in range(DG):
            pltpu.async_copy(ones_v, deg_sh.at[idx_v.at[j]], sem, add=True)
        for j in range(DG):
            pltpu.make_async_copy(ones_v, deg_sh.at[idx_v.at[j]], sem).wait()

    load(0, idxa_v)

    def body(i, carry):
        g0 = 2 * i
        g1 = g0 + 1
        g2 = g0 + 2

        @pl.when(g1 < gpw)
        def _():
            load(g1, idxb_v)

        group(idxa_v)

        @pl.when(g2 < gpw)
        def _():
            load(g2, idxa_v)

        @pl.when(g1 < gpw)
        def _():
            group(idxb_v)

        return carry

    lax.fori_loop(0, (gpw + 1) // 2, body, 0)
    plsc.subcore_barrier()
    pltpu.sync_copy(deg_sh.at[pl.ds(sid * spt, spt)], slab_v)
    pltpu.sync_copy(slab_v, out_hbm.at[pl.ds(cid * dpad + sid * spt, spt)])


def _agg_body(src_hbm, dst_hbm, g_hbm, zeros_hbm, out_hbm,
              src0_v, dst0_v, rows0_v, sem0,
              src1_v, dst1_v, rows1_v, sem1, acc_sh):
    cid = lax.axis_index("c")
    sid = lax.axis_index("s")
    wid = cid * NS + sid
    n = acc_sh.shape[0]
    _part_copy(sid, n, lambda o, sz: pltpu.sync_copy(
        zeros_hbm.at[pl.ds(o, sz)], acc_sh.at[pl.ds(o, sz)]))
    plsc.subcore_barrier()

    nchunks = src_hbm.shape[0] // CH
    bufs = ((src0_v, dst0_v, rows0_v, sem0), (src1_v, dst1_v, rows1_v, sem1))

    def gstart(c, b):
        src_v, dst_v, rows_v, sem = bufs[b]
        pltpu.sync_copy(src_hbm.at[pl.ds(c * CH, CH)], src_v)
        pltpu.sync_copy(dst_hbm.at[pl.ds(c * CH, CH)], dst_v)
        pltpu.async_copy(g_hbm.at[src_v], rows_v, sem)

    def finish(b):
        src_v, dst_v, rows_v, sem = bufs[b]
        pltpu.make_async_copy(g_hbm.at[src_v], rows_v, sem).wait()
        pltpu.sync_copy(rows_v, acc_sh.at[dst_v], add=True)

    gstart(wid, 0)

    def body(i, carry):
        c0 = wid + (2 * i) * NW
        c1 = c0 + NW
        c2 = c1 + NW

        @pl.when(c1 < nchunks)
        def _():
            gstart(c1, 1)

        @pl.when(c0 < nchunks)
        def _():
            finish(0)

        @pl.when(c2 < nchunks)
        def _():
            gstart(c2, 0)

        @pl.when(c1 < nchunks)
        def _():
            finish(1)

        return carry

    lax.fori_loop(0, (nchunks // NW + 2) // 2, body, 0)
    plsc.subcore_barrier()
    _part_copy(sid, n, lambda o, sz: pltpu.sync_copy(
        acc_sh.at[pl.ds(o, sz)], out_hbm.at[cid, pl.ds(o, sz)]))


def _sc_degree(dst, ones, dpad):
    call = pl.kernel(
        _deg_body,
        out_type=jax.ShapeDtypeStruct((NC * dpad,), jnp.float32),
        mesh=_sc_mesh(),
        scratch_types=[
            pltpu.VMEM((CH,), jnp.float32),
            pltpu.VMEM((DG, CH), jnp.int32),
            pltpu.VMEM((DG, CH), jnp.int32),
            pltpu.VMEM((dpad // NS,), jnp.float32),
            pltpu.SemaphoreType.DMA,
            pltpu.VMEM_SHARED((dpad,), jnp.float32),
        ],
    )
    return call(dst, ones)


def _sc_aggregate(src, dst, g, zeros, n, d):
    call = pl.kernel(
        _agg_body,
        out_type=jax.ShapeDtypeStruct((NC, n, d), jnp.float32),
        mesh=_sc_mesh(),
        scratch_types=[
            pltpu.VMEM((CH,), jnp.int32),
            pltpu.VMEM((CH,), jnp.int32),
            pltpu.VMEM((CH, d), jnp.float32),
            pltpu.SemaphoreType.DMA,
            pltpu.VMEM((CH,), jnp.int32),
            pltpu.VMEM((CH,), jnp.int32),
            pltpu.VMEM((CH, d), jnp.float32),
            pltpu.SemaphoreType.DMA,
            pltpu.VMEM_SHARED((n, d), jnp.float32),
        ],
    )
    return call(src, dst, g, zeros)


def _lin1_body(x_ref, w_ref, d0_ref, d1_ref, g_ref, dv_ref):
    deg = d0_ref[...] + d1_ref[...] + 1.0
    dinv = lax.rsqrt(deg)
    dv_ref[...] = jnp.broadcast_to(dinv, dv_ref.shape)
    u = jnp.dot(x_ref[...], w_ref[...], preferred_element_type=jnp.float32)
    g_ref[...] = dinv * u


def _mid_body(a0_ref, a1_ref, g_ref, dv_ref, b_ref, w_ref, o_ref):
    dinv = dv_ref[:, 0:1]
    h = dinv * (a0_ref[...] + a1_ref[...] + g_ref[...]) + b_ref[...]
    h = jnp.maximum(h, 0.0)
    o_ref[...] = dinv * jnp.dot(h, w_ref[...], preferred_element_type=jnp.float32)


def _out_body(a0_ref, a1_ref, g_ref, dv_ref, b_ref, w_ref, b3_ref, o_ref):
    dinv = dv_ref[:, 0:1]
    h = dinv * (a0_ref[...] + a1_ref[...] + g_ref[...]) + b_ref[...]
    h = jnp.maximum(h, 0.0)
    o_ref[...] = (
        jnp.dot(h, w_ref[...], preferred_element_type=jnp.float32) + b3_ref[...]
    )


def _row_spec(br, d):
    return pl.BlockSpec((br, d), lambda i: (i, 0))


def _rep_spec(shape):
    return pl.BlockSpec(shape, lambda i: (0,) * len(shape))


DVW = 8


def _tc_lin1(x, w, d0, d1, br):
    n, d = x.shape
    h = w.shape[1]
    return pl.pallas_call(
        _lin1_body,
        grid=(n // br,),
        in_specs=[
            _row_spec(br, d),
            _rep_spec(w.shape),
            _row_spec(br, 1),
            _row_spec(br, 1),
        ],
        out_specs=[_row_spec(br, h), _row_spec(br, DVW)],
        out_shape=[
            jax.ShapeDtypeStruct((n, h), jnp.float32),
            jax.ShapeDtypeStruct((n, DVW), jnp.float32),
        ],
    )(x, w, d0, d1)


def _tc_mid(a0, a1, g, dv, b, w, br):
    n, h = g.shape
    h2 = w.shape[1]
    return pl.pallas_call(
        _mid_body,
        grid=(n // br,),
        in_specs=[
            _row_spec(br, h),
            _row_spec(br, h),
            _row_spec(br, h),
            _row_spec(br, DVW),
            _rep_spec(b.shape),
            _rep_spec(w.shape),
        ],
        out_specs=_row_spec(br, h2),
        out_shape=jax.ShapeDtypeStruct((n, h2), jnp.float32),
    )(a0, a1, g, dv, b, w)


def _tc_out(a0, a1, g, dv, b, w, b3, br):
    n, h = g.shape
    c = w.shape[1]
    return pl.pallas_call(
        _out_body,
        grid=(n // br,),
        in_specs=[
            _row_spec(br, h),
            _row_spec(br, h),
            _row_spec(br, h),
            _row_spec(br, DVW),
            _rep_spec(b.shape),
            _rep_spec(w.shape),
            _rep_spec(b3.shape),
        ],
        out_specs=_row_spec(br, c),
        out_shape=jax.ShapeDtypeStruct((n, c), jnp.float32),
    )(a0, a1, g, dv, b, w, b3)


@jax.jit
def kernel(x, edge_index, W1, b1, W2, b2, W3, b3):
    n, d = x.shape
    h = W1.shape[1]
    c = W3.shape[1]
    e = edge_index.shape[1]
    assert n % NS == 0 and e % CH == 0 and d % 128 == 0 and h % 128 == 0

    br = 2000 if n % 2000 == 0 else max(
        bb for bb in (8, 16, 40, 200, 400, 1000) if n % bb == 0
    )

    src = edge_index[0].astype(jnp.int32)
    dst = edge_index[1].astype(jnp.int32)

    ones = jnp.ones((CH,), jnp.float32)
    zeros_acc = jnp.zeros((n, h), jnp.float32)

    spt = (-(-n // NS) + 15) // 16 * 16
    if NS * spt == n:
        spt += 16
    dpad = NS * spt

    nchunks = e // CH
    gpw = -(-nchunks // (NW * DG))
    rows = NW * gpw * DG
    pad = rows * CH - e
    dst2d = jnp.concatenate(
        [dst, n + jnp.arange(pad, dtype=jnp.int32) % (dpad - n)]
    ).reshape(rows, CH)

    degf = _sc_degree(dst2d, ones, dpad)
    d2 = degf.reshape(NC, dpad)
    d0 = d2[0, :n].reshape(n, 1)
    d1 = d2[1, :n].reshape(n, 1)

    b1r = b1.reshape(1, h)
    b2r = b2.reshape(1, h)

    wpad = 128
    W3p = jnp.zeros((h, wpad), jnp.float32).at[:, :c].set(W3)
    b3p = jnp.zeros((1, wpad), jnp.float32).at[:, :c].set(b3)

    g1, dv = _tc_lin1(x, W1, d0, d1, br)
    acc1 = _sc_aggregate(src, dst, g1, zeros_acc, n, h)
    g2 = _tc_mid(acc1[0], acc1[1], g1, dv, b1r, W2, br)
    acc2 = _sc_aggregate(src, dst, g2, zeros_acc, n, h)
    outp = _tc_out(acc2[0], acc2[1], g2, dv, b2r, W3p, b3p, br)
    return outp[:, :c]


# speedup vs baseline: 24.6854x; 1.0012x over previous
import jax
import jax.numpy as jnp
from jax import lax
from jax.experimental import pallas as pl
from jax.experimental.pallas import tpu as pltpu
from jax.experimental.pallas import tpu_sc as plsc

NC = 2
NS = 16
NW = NC * NS
CH = 128


def _sc_mesh():
    return plsc.VectorSubcoreMesh(
        core_axis_name="c", subcore_axis_name="s", num_cores=NC, num_subcores=NS
    )


def _part_copy(sid, n, copy_slice):
    rpt = (n // NS) // 8 * 8
    rem = n - NS * rpt
    copy_slice(sid * rpt, rpt)
    if rem:
        @pl.when(sid == 0)
        def _():
            copy_slice(NS * rpt, rem)


DG = 8


def _deg_body(dst_hbm, ones_hbm, out_hbm,
              ones_v, idxa_v, idxb_v, slab_v, sem, deg_sh):
    cid = lax.axis_index("c")
    sid = lax.axis_index("s")
    wid = cid * NS + sid
    dpad = deg_sh.shape[0]
    spt = dpad // NS
    for i in range(spt // 16):
        slab_v[pl.ds(i * 16, 16)] = jnp.zeros((16,), jnp.float32)
    pltpu.sync_copy(slab_v, deg_sh.at[pl.ds(sid * spt, spt)])
    pltpu.sync_copy(ones_hbm, ones_v)
    plsc.subcore_barrier()

    gpw = dst_hbm.shape[0] // (NW * DG)
    row0 = wid * gpw * DG

    def load(g, idx_v):
        pltpu.sync_copy(dst_hbm.at[pl.ds(row0 + g * DG, DG)], idx_v)

    def group(idx_v):
        for j in range(DG):
            pltpu.async_copy(ones_v, deg_sh.at[idx_v.at[j]], sem, add=True)
        for j in range(DG):
            pltpu.make_async_copy(ones_v, deg_sh.at[idx_v.at[j]], sem).wait()

    load(0, idxa_v)

    def body(i, carry):
        g0 = 2 * i
        g1 = g0 + 1
        g2 = g0 + 2

        @pl.when(g1 < gpw)
        def _():
            load(g1, idxb_v)

        group(idxa_v)

        @pl.when(g2 < gpw)
        def _():
            load(g2, idxa_v)

        @pl.when(g1 < gpw)
        def _():
            group(idxb_v)

        return carry

    lax.fori_loop(0, (gpw + 1) // 2, body, 0)
    plsc.subcore_barrier()
    pltpu.sync_copy(deg_sh.at[pl.ds(sid * spt, spt)], slab_v)
    pltpu.sync_copy(slab_v, out_hbm.at[pl.ds(cid * dpad + sid * spt, spt)])


def _agg_body(src_hbm, dst_hbm, g_hbm, zeros_hbm, out_hbm,
              src0_v, dst0_v, rows0_v, sem0,
              src1_v, dst1_v, rows1_v, sem1, acc_sh):
    cid = lax.axis_index("c")
    sid = lax.axis_index("s")
    wid = cid * NS + sid
    n = acc_sh.shape[0]
    _part_copy(sid, n, lambda o, sz: pltpu.sync_copy(
        zeros_hbm.at[pl.ds(o, sz)], acc_sh.at[pl.ds(o, sz)]))
    plsc.subcore_barrier()

    nchunks = src_hbm.shape[0] // CH
    bufs = ((src0_v, dst0_v, rows0_v, sem0), (src1_v, dst1_v, rows1_v, sem1))

    def gstart(c, b):
        src_v, dst_v, rows_v, sem = bufs[b]
        pltpu.sync_copy(src_hbm.at[pl.ds(c * CH, CH)], src_v)
        pltpu.sync_copy(dst_hbm.at[pl.ds(c * CH, CH)], dst_v)
        pltpu.async_copy(g_hbm.at[src_v], rows_v, sem)

    def finish(b):
        src_v, dst_v, rows_v, sem = bufs[b]
        pltpu.make_async_copy(g_hbm.at[src_v], rows_v, sem).wait()
        pltpu.sync_copy(rows_v, acc_sh.at[dst_v], add=True)

    gstart(wid, 0)

    def body(i, carry):
        c0 = wid + (2 * i) * NW
        c1 = c0 + NW
        c2 = c1 + NW

        @pl.when(c1 < nchunks)
        def _():
            gstart(c1, 1)

        @pl.when(c0 < nchunks)
        def _():
            finish(0)

        @pl.when(c2 < nchunks)
        def _():
            gstart(c2, 0)

        @pl.when(c1 < nchunks)
        def _():
            finish(1)

        return carry

    lax.fori_loop(0, (nchunks // NW + 2) // 2, body, 0)
    plsc.subcore_barrier()
    _part_copy(sid, n, lambda o, sz: pltpu.sync_copy(
        acc_sh.at[pl.ds(o, sz)], out_hbm.at[cid, pl.ds(o, sz)]))


def _sc_degree(dst, ones, dpad):
    call = pl.kernel(
        _deg_body,
        out_type=jax.ShapeDtypeStruct((NC * dpad,), jnp.float32),
        mesh=_sc_mesh(),
        scratch_types=[
            pltpu.VMEM((CH,), jnp.float32),
            pltpu.VMEM((DG, CH), jnp.int32),
            pltpu.VMEM((DG, CH), jnp.int32),
            pltpu.VMEM((dpad // NS,), jnp.float32),
            pltpu.SemaphoreType.DMA,
            pltpu.VMEM_SHARED((dpad,), jnp.float32),
        ],
    )
    return call(dst, ones)


def _sc_aggregate(src, dst, g, zeros, n, d):
    call = pl.kernel(
        _agg_body,
        out_type=jax.ShapeDtypeStruct((NC, n, d), jnp.float32),
        mesh=_sc_mesh(),
        scratch_types=[
            pltpu.VMEM((CH,), jnp.int32),
            pltpu.VMEM((CH,), jnp.int32),
            pltpu.VMEM((CH, d), jnp.float32),
            pltpu.SemaphoreType.DMA,
            pltpu.VMEM((CH,), jnp.int32),
            pltpu.VMEM((CH,), jnp.int32),
            pltpu.VMEM((CH, d), jnp.float32),
            pltpu.SemaphoreType.DMA,
            pltpu.VMEM_SHARED((n, d), jnp.float32),
        ],
    )
    return call(src, dst, g, zeros)


def _lin1_body(x_ref, w_ref, d0_ref, d1_ref, g_ref, dv_ref):
    deg = d0_ref[...] + d1_ref[...] + 1.0
    dinv = lax.rsqrt(deg)
    dv_ref[...] = jnp.broadcast_to(dinv, dv_ref.shape)
    u = jnp.dot(x_ref[...], w_ref[...], preferred_element_type=jnp.float32)
    g_ref[...] = dinv * u


def _mid_body(a0_ref, a1_ref, g_ref, dv_ref, b_ref, w_ref, o_ref):
    dinv = dv_ref[:, 0:1]
    h = dinv * (a0_ref[...] + a1_ref[...] + g_ref[...]) + b_ref[...]
    h = jnp.maximum(h, 0.0)
    o_ref[...] = dinv * jnp.dot(h, w_ref[...], preferred_element_type=jnp.float32)


def _out_body(a0_ref, a1_ref, g_ref, dv_ref, b_ref, w_ref, b3_ref, o_ref):
    dinv = dv_ref[:, 0:1]
    h = dinv * (a0_ref[...] + a1_ref[...] + g_ref[...]) + b_ref[...]
    h = jnp.maximum(h, 0.0)
    o_ref[...] = (
        jnp.dot(h, w_ref[...], preferred_element_type=jnp.float32) + b3_ref[...]
    )


def _row_spec(br, d):
    return pl.BlockSpec((br, d), lambda i: (i, 0))


def _rep_spec(shape):
    return pl.BlockSpec(shape, lambda i: (0,) * len(shape))


DVW = 8


def _tc_lin1(x, w, d0, d1, br):
    n, d = x.shape
    h = w.shape[1]
    return pl.pallas_call(
        _lin1_body,
        grid=(n // br,),
        in_specs=[
            _row_spec(br, d),
            _rep_spec(w.shape),
            _row_spec(br, 1),
            _row_spec(br, 1),
        ],
        out_specs=[_row_spec(br, h), _row_spec(br, DVW)],
        out_shape=[
            jax.ShapeDtypeStruct((n, h), jnp.float32),
            jax.ShapeDtypeStruct((n, DVW), jnp.float32),
        ],
    )(x, w, d0, d1)


def _tc_mid(a0, a1, g, dv, b, w, br):
    n, h = g.shape
    h2 = w.shape[1]
    return pl.pallas_call(
        _mid_body,
        grid=(n // br,),
        in_specs=[
            _row_spec(br, h),
            _row_spec(br, h),
            _row_spec(br, h),
            _row_spec(br, DVW),
            _rep_spec(b.shape),
            _rep_spec(w.shape),
        ],
        out_specs=_row_spec(br, h2),
        out_shape=jax.ShapeDtypeStruct((n, h2), jnp.float32),
    )(a0, a1, g, dv, b, w)


def _tc_out(a0, a1, g, dv, b, w, b3, br):
    n, h = g.shape
    c = w.shape[1]
    return pl.pallas_call(
        _out_body,
        grid=(n // br,),
        in_specs=[
            _row_spec(br, h),
            _row_spec(br, h),
            _row_spec(br, h),
            _row_spec(br, DVW),
            _rep_spec(b.shape),
            _rep_spec(w.shape),
            _rep_spec(b3.shape),
        ],
        out_specs=_row_spec(br, c),
        out_shape=jax.ShapeDtypeStruct((n, c), jnp.float32),
    )(a0, a1, g, dv, b, w, b3)


@jax.jit
def kernel(x, edge_index, W1, b1, W2, b2, W3, b3):
    n, d = x.shape
    h = W1.shape[1]
    c = W3.shape[1]
    e = edge_index.shape[1]
    assert n % NS == 0 and e % CH == 0 and d % 128 == 0 and h % 128 == 0

    br = 2000 if n % 2000 == 0 else max(
        bb for bb in (8, 16, 40, 200, 400, 1000) if n % bb == 0
    )

    src = edge_index[0].astype(jnp.int32)
    dst = edge_index[1].astype(jnp.int32)

    ones = jnp.ones((CH,), jnp.float32)
    zeros_acc = jnp.zeros((n, h), jnp.float32)

    spt = (-(-n // NS) + 15) // 16 * 16
    if NS * spt == n:
        spt += 16
    dpad = NS * spt

    nchunks = e // CH
    gpw = -(-nchunks // (NW * DG))
    rows = NW * gpw * DG
    pad = rows * CH - e
    dst2d = jnp.concatenate(
        [dst, n + jnp.arange(pad, dtype=jnp.int32) % (dpad - n)]
    ).reshape(rows, CH)

    degf = _sc_degree(dst2d, ones, dpad)
    d2 = degf.reshape(NC, dpad)
    d0 = d2[0, :n].reshape(n, 1)
    d1 = d2[1, :n].reshape(n, 1)

    b1r = b1.reshape(1, h)
    b2r = b2.reshape(1, h)

    wpad = 128
    W3p = jnp.zeros((h, wpad), jnp.float32).at[:, :c].set(W3)
    b3p = jnp.zeros((1, wpad), jnp.float32).at[:, :c].set(b3)

    g1, dv = _tc_lin1(x, W1, d0, d1, br)
    acc1 = _sc_aggregate(src, dst, g1, zeros_acc, n, h)
    g2 = _tc_mid(acc1[0], acc1[1], g1, dv, b1r, W2, br)
    acc2 = _sc_aggregate(src, dst, g2, zeros_acc, n, h)
    outp = _tc_out(acc2[0], acc2[1], g2, dv, b2r, W3p, b3p, br)
    return outp[:, :c]
